# baseline (speedup 1.0000x reference)
import functools

import jax
import jax.numpy as jnp
from jax import lax
from jax.experimental import pallas as pl
from jax.experimental.pallas import tpu as pltpu
from jax.experimental.pallas import tpu_sc as plsc

N = 10000
E = 320000
D = 128
HD = 64
OUT = 64

NC = 2
NS = 16
NW = NC * NS

C = 128
KC = (E + NS * C - 1) // (NS * C)
KC = 160
E_PAD = NS * KC * C
N_ACC = 10112
ROWS_PER_TILE = N_ACC // NS

DEG_OFF = N + 240
DEG_LEN = 2 * DEG_OFF
DEG_PER_TILE = DEG_LEN // NS
DEG_K = 2 * E_PAD // (NW * C)

_MESH = plsc.VectorSubcoreMesh(
    core_axis_name="c", subcore_axis_name="s", num_cores=NC, num_subcores=NS
)


@functools.partial(
    pl.kernel,
    out_type=jax.ShapeDtypeStruct((NC, DEG_LEN), jnp.float32),
    mesh=_MESH,
    scratch_types=[
        pltpu.VMEM((DEG_K, C), jnp.int32),
        pltpu.VMEM((C,), jnp.float32),
        pltpu.VMEM((DEG_PER_TILE,), jnp.float32),
        pltpu.VMEM_SHARED((DEG_LEN,), jnp.float32),
    ],
)
def _deg_kernel(idx_hbm, out_hbm, idx_v, ones_v, stage_v, acc):
    c = lax.axis_index("c")
    s = lax.axis_index("s")
    w = c * NS + s

    one = jnp.ones((16,), jnp.float32)
    zero = jnp.zeros((16,), jnp.float32)
    for j in range(C // 16):
        ones_v[pl.ds(j * 16, 16)] = one

    @pl.loop(0, DEG_PER_TILE // 16)
    def _(r):
        stage_v[pl.ds(r * 16, 16)] = zero

    pltpu.sync_copy(stage_v, acc.at[pl.ds(s * DEG_PER_TILE, DEG_PER_TILE)])
    pltpu.sync_copy(idx_hbm.at[w], idx_v)
    plsc.subcore_barrier()

    @pl.loop(0, DEG_K)
    def _(j):
        pltpu.sync_copy(ones_v, acc.at[idx_v.at[j]], add=True)

    plsc.subcore_barrier()
    pltpu.sync_copy(acc.at[pl.ds(s * DEG_PER_TILE, DEG_PER_TILE)], stage_v)
    pltpu.sync_copy(stage_v, out_hbm.at[c, pl.ds(s * DEG_PER_TILE, DEG_PER_TILE)])


NBUF = 5
GD = 4


@functools.partial(
    pl.kernel,
    out_type=jax.ShapeDtypeStruct((NC, N_ACC, HD), jnp.float32),
    mesh=_MESH,
    scratch_types=[
        pltpu.VMEM((KC, C), jnp.int32),
        pltpu.VMEM((KC, C), jnp.int32),
        [pltpu.VMEM((C, HD), jnp.float32)] * NBUF,
        [pltpu.SemaphoreType.DMA] * NBUF,
        pltpu.VMEM_SHARED((N_ACC, HD), jnp.float32),
    ],
    compiler_params=pltpu.CompilerParams(use_tc_tiling_on_sc=False),
)
def _prop_kernel(table_hbm, src_hbm, dst_hbm, out_hbm,
                 idx_s, idx_d, rows, sem_g, acc):
    c = lax.axis_index("c")
    s = lax.axis_index("s")
    tbl_c = table_hbm.at[c]

    zero = jnp.zeros((16,), jnp.float32)

    @pl.loop(0, C)
    def _(r):
        for j in range(HD // 16):
            rows[0][r, pl.ds(j * 16, 16)] = zero

    base = s * ROWS_PER_TILE
    n_full, rem = divmod(ROWS_PER_TILE, C)
    for i in range(n_full):
        pltpu.sync_copy(rows[0], acc.at[pl.ds(base + i * C, C)])
    if rem:
        pltpu.sync_copy(rows[0].at[pl.ds(0, rem)],
                        acc.at[pl.ds(base + n_full * C, rem)])
    pltpu.sync_copy(src_hbm.at[s], idx_s)
    pltpu.sync_copy(dst_hbm.at[s], idx_d)
    plsc.subcore_barrier()

    def _gather(chunk, b):
        pltpu.async_copy(tbl_c.at[idx_s.at[chunk]], rows[b], sem_g[b])

    def _wait_gather(chunk, b):
        pltpu.make_async_copy(tbl_c.at[idx_s.at[chunk]], rows[b],
                              sem_g[b]).wait()

    for j in range(GD):
        _gather(j, j)

    @pl.loop(0, KC, step=NBUF)
    def _(g):
        for j in range(NBUF):
            _wait_gather(g + j, j)
            pltpu.sync_copy(rows[j], acc.at[idx_d.at[g + j]], add=True)

            @pl.when(g + j + GD < KC)
            def _():
                _gather(g + j + GD, (j + GD) % NBUF)

    plsc.subcore_barrier()

    for i in range(n_full):
        pltpu.sync_copy(acc.at[pl.ds(base + i * C, C)], rows[0])
        pltpu.sync_copy(rows[0], out_hbm.at[c, pl.ds(base + i * C, C)])
    if rem:
        pltpu.sync_copy(acc.at[pl.ds(base + n_full * C, rem)],
                        rows[0].at[pl.ds(0, rem)])
        pltpu.sync_copy(rows[0].at[pl.ds(0, rem)],
                        out_hbm.at[c, pl.ds(base + n_full * C, rem)])


_RB = 1000
_GRID = N // _RB


def _norm_from(degp_ref):
    d = degp_ref[0] + degp_ref[1]
    return jnp.where(d > 0.0, lax.rsqrt(d), 0.0)


def _split_store(o_ref, v):
    o_ref[0] = v[:, :HD]
    o_ref[1] = v[:, HD:]


def _scale_body(x_ref, dout_ref, o_ref):
    _split_store(o_ref, x_ref[...] * _norm_from(dout_ref))


def _layer_body(p_ref, din_ref, dout_ref, w_ref, b_ref, o_ref):
    agg = jnp.concatenate([p_ref[0], p_ref[1]], axis=1) * _norm_from(din_ref)
    h = jnp.dot(agg, w_ref[...], preferred_element_type=jnp.float32)
    h = jax.nn.sigmoid(h + b_ref[...])
    _split_store(o_ref, h * _norm_from(dout_ref))


def _final_body(q_ref, din_ref, w2_ref, b2_ref, wfc_ref, bfc_ref, o_ref):
    agg = jnp.concatenate([q_ref[0], q_ref[1]], axis=1) * _norm_from(din_ref)
    h = jnp.dot(agg, w2_ref[...], preferred_element_type=jnp.float32)
    h = jax.nn.sigmoid(h + b2_ref[...])
    o_ref[...] = jnp.dot(h, wfc_ref[...],
                         preferred_element_type=jnp.float32) + bfc_ref[...]


def _deg_spec():
    return pl.BlockSpec((2, _RB, 1), lambda i: (0, i, 0))


def _half_spec():
    return pl.BlockSpec((2, _RB, HD), lambda i: (0, i, 0))


def _full_spec(shape):
    return pl.BlockSpec(shape, lambda i: tuple(0 for _ in shape))


_scale_call = pl.pallas_call(
    _scale_body,
    grid=(_GRID,),
    in_specs=[
        pl.BlockSpec((_RB, D), lambda i: (i, 0)),
        _deg_spec(),
    ],
    out_specs=_half_spec(),
    out_shape=jax.ShapeDtypeStruct((2, N, HD), jnp.float32),
)

_layer_call = pl.pallas_call(
    _layer_body,
    grid=(_GRID,),
    in_specs=[
        _half_spec(),
        _deg_spec(),
        _deg_spec(),
        _full_spec((D, D)),
        _full_spec((1, D)),
    ],
    out_specs=_half_spec(),
    out_shape=jax.ShapeDtypeStruct((2, N, HD), jnp.float32),
)

_final_call = pl.pallas_call(
    _final_body,
    grid=(_GRID,),
    in_specs=[
        _half_spec(),
        _deg_spec(),
        _full_spec((D, D)),
        _full_spec((1, D)),
        _full_spec((D, OUT)),
        _full_spec((1, OUT)),
    ],
    out_specs=pl.BlockSpec((_RB, OUT), lambda i: (i, 0)),
    out_shape=jax.ShapeDtypeStruct((N, OUT), jnp.float32),
)


def kernel(x, edge_index, W1, b1, W2, b2, Wfc, bfc):
    src = edge_index[0].astype(jnp.int32)
    dst = edge_index[1].astype(jnp.int32)

    pad = E_PAD - E
    ar = jnp.arange(pad, dtype=jnp.int32)
    src_p = jnp.concatenate([src, (ar * 131) % N]).reshape(NS, KC, C)
    dst_p = jnp.concatenate([dst, N + (ar % NS)]).reshape(NS, KC, C)
    deg_idx = jnp.concatenate([
        src, N + (ar % 64),
        dst + DEG_OFF, DEG_OFF + N + (ar % 64),
    ]).reshape(NW, DEG_K, C)

    degp = _deg_kernel(deg_idx)
    dout = degp[:, :N].reshape(NC, N, 1)
    din = degp[:, DEG_OFF:DEG_OFF + N].reshape(NC, N, 1)

    xs = _scale_call(x, dout)
    p = _prop_kernel(xs, src_p, dst_p)
    t2 = _layer_call(p, din, dout, W1, b1.reshape(1, D))
    q = _prop_kernel(t2, src_p, dst_p)
    out = _final_call(q, din, W2, b2.reshape(1, D),
                      Wfc, bfc.reshape(1, OUT))
    return out


# speedup vs baseline: 1.1462x; 1.1462x over previous
import functools

import jax
import jax.numpy as jnp
from jax import lax
from jax.experimental import pallas as pl
from jax.experimental.pallas import tpu as pltpu
from jax.experimental.pallas import tpu_sc as plsc

N = 10000
E = 320000
D = 128
HD = 64
OUT = 64

NC = 2
NS = 16
NW = NC * NS

C = 128
KC = (E + NS * C - 1) // (NS * C)
KC = 160
E_PAD = NS * KC * C
N_ACC = 10112
ROWS_PER_TILE = N_ACC // NS

DEG_OFF = N + 240
DEG_LEN = 2 * DEG_OFF
DEG_PER_TILE = DEG_LEN // NS
DEG_K = 2 * E_PAD // (NW * C)

_MESH = plsc.VectorSubcoreMesh(
    core_axis_name="c", subcore_axis_name="s", num_cores=NC, num_subcores=NS
)


@functools.partial(
    pl.kernel,
    out_type=jax.ShapeDtypeStruct((NC, DEG_LEN), jnp.float32),
    mesh=_MESH,
    scratch_types=[
        pltpu.VMEM((DEG_K, C), jnp.int32),
        pltpu.VMEM((C,), jnp.float32),
        pltpu.VMEM((DEG_PER_TILE,), jnp.float32),
        pltpu.VMEM_SHARED((DEG_LEN,), jnp.float32),
    ],
)
def _deg_kernel(idx_hbm, out_hbm, idx_v, ones_v, stage_v, acc):
    c = lax.axis_index("c")
    s = lax.axis_index("s")
    w = c * NS + s

    one = jnp.ones((16,), jnp.float32)
    zero = jnp.zeros((16,), jnp.float32)
    for j in range(C // 16):
        ones_v[pl.ds(j * 16, 16)] = one

    @pl.loop(0, DEG_PER_TILE // 16)
    def _(r):
        stage_v[pl.ds(r * 16, 16)] = zero

    pltpu.sync_copy(stage_v, acc.at[pl.ds(s * DEG_PER_TILE, DEG_PER_TILE)])
    pltpu.sync_copy(idx_hbm.at[w], idx_v)
    plsc.subcore_barrier()

    @pl.loop(0, DEG_K)
    def _(j):
        pltpu.sync_copy(ones_v, acc.at[idx_v.at[j]], add=True)

    plsc.subcore_barrier()
    pltpu.sync_copy(acc.at[pl.ds(s * DEG_PER_TILE, DEG_PER_TILE)], stage_v)
    pltpu.sync_copy(stage_v, out_hbm.at[c, pl.ds(s * DEG_PER_TILE, DEG_PER_TILE)])


NBUF = 5
GD = 4


@functools.partial(
    pl.kernel,
    out_type=jax.ShapeDtypeStruct((N_ACC, D), jnp.float32),
    mesh=_MESH,
    scratch_types=[
        pltpu.VMEM((KC, C), jnp.int32),
        pltpu.VMEM((KC, C), jnp.int32),
        [pltpu.VMEM((C, HD), jnp.float32)] * NBUF,
        [pltpu.SemaphoreType.DMA] * NBUF,
        pltpu.VMEM_SHARED((N_ACC, HD), jnp.float32),
    ],
    compiler_params=pltpu.CompilerParams(use_tc_tiling_on_sc=False),
)
def _prop_kernel(table_hbm, src_hbm, dst_hbm, out_hbm,
                 idx_s, idx_d, rows, sem_g, acc):
    c = lax.axis_index("c")
    s = lax.axis_index("s")

    zero = jnp.zeros((16,), jnp.float32)

    @pl.loop(0, C)
    def _(r):
        for j in range(HD // 16):
            rows[0][r, pl.ds(j * 16, 16)] = zero

    base = s * ROWS_PER_TILE
    n_full, rem = divmod(ROWS_PER_TILE, C)
    for i in range(n_full):
        pltpu.sync_copy(rows[0], acc.at[pl.ds(base + i * C, C)])
    if rem:
        pltpu.sync_copy(rows[0].at[pl.ds(0, rem)],
                        acc.at[pl.ds(base + n_full * C, rem)])
    pltpu.sync_copy(src_hbm.at[s], idx_s)
    pltpu.sync_copy(dst_hbm.at[s], idx_d)

    @pl.loop(0, KC)
    def _(k):
        for jj in range(C // 16):
            v = idx_s[k, pl.ds(jj * 16, 16)]
            idx_s[k, pl.ds(jj * 16, 16)] = v + v + c

    plsc.subcore_barrier()

    def _gather(chunk, b):
        pltpu.async_copy(table_hbm.at[idx_s.at[chunk]], rows[b], sem_g[b])

    def _wait_gather(chunk, b):
        pltpu.make_async_copy(table_hbm.at[idx_s.at[chunk]], rows[b],
                              sem_g[b]).wait()

    for j in range(GD):
        _gather(j, j)

    @pl.loop(0, KC, step=NBUF)
    def _(g):
        for j in range(NBUF):
            _wait_gather(g + j, j)
            pltpu.sync_copy(rows[j], acc.at[idx_d.at[g + j]], add=True)

            @pl.when(g + j + GD < KC)
            def _():
                _gather(g + j + GD, (j + GD) % NBUF)

    plsc.subcore_barrier()

    col = c * HD
    for i in range(n_full):
        pltpu.sync_copy(acc.at[pl.ds(base + i * C, C)], rows[0])
        pltpu.sync_copy(rows[0],
                        out_hbm.at[pl.ds(base + i * C, C), pl.ds(col, HD)])
    if rem:
        pltpu.sync_copy(acc.at[pl.ds(base + n_full * C, rem)],
                        rows[0].at[pl.ds(0, rem)])
        pltpu.sync_copy(rows[0].at[pl.ds(0, rem)],
                        out_hbm.at[pl.ds(base + n_full * C, rem),
                                   pl.ds(col, HD)])


_RB = 1000
_GRID = N // _RB


def _norm_from(degp_ref):
    d = degp_ref[0] + degp_ref[1]
    return jnp.where(d > 0.0, lax.rsqrt(d), 0.0)


def _scale_body(x_ref, dout_ref, o_ref):
    o_ref[...] = x_ref[...] * _norm_from(dout_ref)


def _layer_body(p_ref, din_ref, dout_ref, w_ref, b_ref, o_ref):
    agg = p_ref[...] * _norm_from(din_ref)
    h = jnp.dot(agg, w_ref[...], preferred_element_type=jnp.float32)
    h = jax.nn.sigmoid(h + b_ref[...])
    o_ref[...] = h * _norm_from(dout_ref)


def _final_body(q_ref, din_ref, w2_ref, b2_ref, wfc_ref, bfc_ref, o_ref):
    agg = q_ref[...] * _norm_from(din_ref)
    h = jnp.dot(agg, w2_ref[...], preferred_element_type=jnp.float32)
    h = jax.nn.sigmoid(h + b2_ref[...])
    o_ref[...] = jnp.dot(h, wfc_ref[...],
                         preferred_element_type=jnp.float32) + bfc_ref[...]


def _deg_spec():
    return pl.BlockSpec((2, _RB, 1), lambda i: (0, i, 0))


def _row_spec(w):
    return pl.BlockSpec((_RB, w), lambda i: (i, 0))


def _full_spec(shape):
    return pl.BlockSpec(shape, lambda i: tuple(0 for _ in shape))


_scale_call = pl.pallas_call(
    _scale_body,
    grid=(_GRID,),
    in_specs=[
        _row_spec(D),
        _deg_spec(),
    ],
    out_specs=_row_spec(D),
    out_shape=jax.ShapeDtypeStruct((N, D), jnp.float32),
)

_layer_call = pl.pallas_call(
    _layer_body,
    grid=(_GRID,),
    in_specs=[
        _row_spec(D),
        _deg_spec(),
        _deg_spec(),
        _full_spec((D, D)),
        _full_spec((1, D)),
    ],
    out_specs=_row_spec(D),
    out_shape=jax.ShapeDtypeStruct((N, D), jnp.float32),
)

_final_call = pl.pallas_call(
    _final_body,
    grid=(_GRID,),
    in_specs=[
        _row_spec(D),
        _deg_spec(),
        _full_spec((D, D)),
        _full_spec((1, D)),
        _full_spec((D, OUT)),
        _full_spec((1, OUT)),
    ],
    out_specs=pl.BlockSpec((_RB, OUT), lambda i: (i, 0)),
    out_shape=jax.ShapeDtypeStruct((N, OUT), jnp.float32),
)


def kernel(x, edge_index, W1, b1, W2, b2, Wfc, bfc):
    src = edge_index[0].astype(jnp.int32)
    dst = edge_index[1].astype(jnp.int32)

    pad = E_PAD - E
    ar = jnp.arange(pad, dtype=jnp.int32)
    src_p = jnp.concatenate([src, (ar * 131) % N]).reshape(NS, KC, C)
    dst_p = jnp.concatenate([dst, N + (ar % NS)]).reshape(NS, KC, C)
    deg_idx = jnp.concatenate([
        src, N + (ar % 64),
        dst + DEG_OFF, DEG_OFF + N + (ar % 64),
    ]).reshape(NW, DEG_K, C)

    degp = _deg_kernel(deg_idx)
    dout = degp[:, :N].reshape(NC, N, 1)
    din = degp[:, DEG_OFF:DEG_OFF + N].reshape(NC, N, 1)

    xs = _scale_call(x, dout)
    p = _prop_kernel(xs.reshape(2 * N, HD), src_p, dst_p)
    t2 = _layer_call(p, din, dout, W1, b1.reshape(1, D))
    q = _prop_kernel(t2.reshape(2 * N, HD), src_p, dst_p)
    out = _final_call(q, din, W2, b2.reshape(1, D),
                      Wfc, bfc.reshape(1, OUT))
    return out


# speedup vs baseline: 1.1770x; 1.0269x over previous
import functools

import jax
import jax.numpy as jnp
import numpy as np
from jax import lax
from jax.experimental import pallas as pl
from jax.experimental.pallas import tpu as pltpu
from jax.experimental.pallas import tpu_sc as plsc

N = 10000
E = 320000
D = 128
HD = 64
OUT = 64

NC = 2
NS = 16
NW = NC * NS

C = 128
KC = (E + NS * C - 1) // (NS * C)
KC = 160
E_PAD = NS * KC * C
N_ACC = 10112
ROWS_PER_TILE = N_ACC // NS

DEG_LEN = N + 240
DEG_PER_TILE = DEG_LEN // NS

_MESH = plsc.VectorSubcoreMesh(
    core_axis_name="c", subcore_axis_name="s", num_cores=NC, num_subcores=NS
)


@functools.partial(
    pl.kernel,
    out_type=jax.ShapeDtypeStruct((NC, DEG_LEN), jnp.float32),
    mesh=_MESH,
    scratch_types=[
        pltpu.VMEM((KC, C), jnp.int32),
        pltpu.VMEM((C,), jnp.float32),
        pltpu.VMEM((DEG_PER_TILE,), jnp.float32),
        pltpu.VMEM_SHARED((DEG_LEN,), jnp.float32),
    ],
    compiler_params=pltpu.CompilerParams(use_tc_tiling_on_sc=False),
)
def _deg_kernel(idx_hbm, out_hbm, idx_v, ones_v, stage_v, acc):
    c = lax.axis_index("c")
    s = lax.axis_index("s")

    one = jnp.ones((16,), jnp.float32)
    zero = jnp.zeros((16,), jnp.float32)
    for j in range(C // 16):
        ones_v[pl.ds(j * 16, 16)] = one

    @pl.loop(0, DEG_PER_TILE // 16)
    def _(r):
        stage_v[pl.ds(r * 16, 16)] = zero

    pltpu.sync_copy(stage_v, acc.at[pl.ds(s * DEG_PER_TILE, DEG_PER_TILE)])
    pltpu.sync_copy(idx_hbm.at[c, s], idx_v)
    plsc.subcore_barrier()

    @pl.loop(0, KC)
    def _(j):
        pltpu.sync_copy(ones_v, acc.at[idx_v.at[j]], add=True)

    plsc.subcore_barrier()
    pltpu.sync_copy(acc.at[pl.ds(s * DEG_PER_TILE, DEG_PER_TILE)], stage_v)
    pltpu.sync_copy(stage_v, out_hbm.at[c, pl.ds(s * DEG_PER_TILE, DEG_PER_TILE)])


NBUF = 5
GD = 4


@functools.partial(
    pl.kernel,
    out_type=jax.ShapeDtypeStruct((N_ACC, D), jnp.float32),
    mesh=_MESH,
    scratch_types=[
        pltpu.VMEM((KC, C), jnp.int32),
        pltpu.VMEM((KC, C), jnp.int32),
        [pltpu.VMEM((C, HD), jnp.float32)] * NBUF,
        [pltpu.SemaphoreType.DMA] * NBUF,
        pltpu.VMEM_SHARED((N_ACC, HD), jnp.float32),
    ],
    compiler_params=pltpu.CompilerParams(use_tc_tiling_on_sc=False),
)
def _prop_kernel(table_hbm, idx_hbm, out_hbm,
                 idx_s, idx_d, rows, sem_g, acc):
    c = lax.axis_index("c")
    s = lax.axis_index("s")

    zero = jnp.zeros((16,), jnp.float32)

    @pl.loop(0, C)
    def _(r):
        for j in range(HD // 16):
            rows[0][r, pl.ds(j * 16, 16)] = zero

    base = s * ROWS_PER_TILE
    n_full, rem = divmod(ROWS_PER_TILE, C)
    for i in range(n_full):
        pltpu.sync_copy(rows[0], acc.at[pl.ds(base + i * C, C)])
    if rem:
        pltpu.sync_copy(rows[0].at[pl.ds(0, rem)],
                        acc.at[pl.ds(base + n_full * C, rem)])
    pltpu.sync_copy(idx_hbm.at[0, s], idx_s)
    pltpu.sync_copy(idx_hbm.at[1, s], idx_d)

    @pl.loop(0, KC)
    def _(k):
        for jj in range(C // 16):
            v = idx_s[k, pl.ds(jj * 16, 16)]
            idx_s[k, pl.ds(jj * 16, 16)] = v + v + c

    plsc.subcore_barrier()

    def _gather(chunk, b):
        pltpu.async_copy(table_hbm.at[idx_s.at[chunk]], rows[b], sem_g[b])

    def _wait_gather(chunk, b):
        pltpu.make_async_copy(table_hbm.at[idx_s.at[chunk]], rows[b],
                              sem_g[b]).wait()

    for j in range(GD):
        _gather(j, j)

    @pl.loop(0, KC, step=NBUF)
    def _(g):
        for j in range(NBUF):
            _wait_gather(g + j, j)
            pltpu.sync_copy(rows[j], acc.at[idx_d.at[g + j]], add=True)

            @pl.when(g + j + GD < KC)
            def _():
                _gather(g + j + GD, (j + GD) % NBUF)

    plsc.subcore_barrier()

    col = c * HD
    for i in range(n_full):
        pltpu.sync_copy(acc.at[pl.ds(base + i * C, C)], rows[0])
        pltpu.sync_copy(rows[0],
                        out_hbm.at[pl.ds(base + i * C, C), pl.ds(col, HD)])
    if rem:
        pltpu.sync_copy(acc.at[pl.ds(base + n_full * C, rem)],
                        rows[0].at[pl.ds(0, rem)])
        pltpu.sync_copy(rows[0].at[pl.ds(0, rem)],
                        out_hbm.at[pl.ds(base + n_full * C, rem),
                                   pl.ds(col, HD)])


_RB = 1000
_GRID = N // _RB


def _norm_from(deg_ref):
    d = deg_ref[...]
    return jnp.where(d > 0.0, lax.rsqrt(d), 0.0)


def _scale_body(x_ref, dout_ref, o_ref):
    o_ref[...] = x_ref[...] * _norm_from(dout_ref)


def _layer_body(p_ref, din_ref, dout_ref, w_ref, b_ref, o_ref):
    agg = p_ref[...] * _norm_from(din_ref)
    h = jnp.dot(agg, w_ref[...], preferred_element_type=jnp.float32)
    h = jax.nn.sigmoid(h + b_ref[...])
    o_ref[...] = h * _norm_from(dout_ref)


def _final_body(q_ref, din_ref, w2_ref, b2_ref, wfc_ref, bfc_ref, o_ref):
    agg = q_ref[...] * _norm_from(din_ref)
    h = jnp.dot(agg, w2_ref[...], preferred_element_type=jnp.float32)
    h = jax.nn.sigmoid(h + b2_ref[...])
    o_ref[...] = jnp.dot(h, wfc_ref[...],
                         preferred_element_type=jnp.float32) + bfc_ref[...]


def _deg_spec():
    return pl.BlockSpec((_RB, 1), lambda i: (i, 0))


def _row_spec(w):
    return pl.BlockSpec((_RB, w), lambda i: (i, 0))


def _full_spec(shape):
    return pl.BlockSpec(shape, lambda i: tuple(0 for _ in shape))


_scale_call = pl.pallas_call(
    _scale_body,
    grid=(_GRID,),
    in_specs=[
        _row_spec(D),
        _deg_spec(),
    ],
    out_specs=_row_spec(D),
    out_shape=jax.ShapeDtypeStruct((N, D), jnp.float32),
)

_layer_call = pl.pallas_call(
    _layer_body,
    grid=(_GRID,),
    in_specs=[
        _row_spec(D),
        _deg_spec(),
        _deg_spec(),
        _full_spec((D, D)),
        _full_spec((1, D)),
    ],
    out_specs=_row_spec(D),
    out_shape=jax.ShapeDtypeStruct((N, D), jnp.float32),
)

_final_call = pl.pallas_call(
    _final_body,
    grid=(_GRID,),
    in_specs=[
        _row_spec(D),
        _deg_spec(),
        _full_spec((D, D)),
        _full_spec((1, D)),
        _full_spec((D, OUT)),
        _full_spec((1, OUT)),
    ],
    out_specs=pl.BlockSpec((_RB, OUT), lambda i: (i, 0)),
    out_shape=jax.ShapeDtypeStruct((N, OUT), jnp.float32),
)


_PAD = E_PAD - E
_PAD_SRC_HIST = np.bincount((np.arange(_PAD) * 131) % N,
                            minlength=N).astype(np.float32)


def kernel(x, edge_index, W1, b1, W2, b2, Wfc, bfc):
    src = edge_index[0].astype(jnp.int32)
    dst = edge_index[1].astype(jnp.int32)

    ar = jnp.arange(_PAD, dtype=jnp.int32)
    idx = jnp.concatenate([
        src, (ar * 131) % N,
        dst, N + (ar % NS),
    ]).reshape(2, NS, KC, C)

    degp = _deg_kernel(idx)
    dout = (degp[0, :N] - jnp.asarray(_PAD_SRC_HIST)).reshape(N, 1)
    din = degp[1, :N].reshape(N, 1)

    xs = _scale_call(x, dout)
    p = _prop_kernel(xs.reshape(2 * N, HD), idx)
    t2 = _layer_call(p, din, dout, W1, b1.reshape(1, D))
    q = _prop_kernel(t2.reshape(2 * N, HD), idx)
    out = _final_call(q, din, W2, b2.reshape(1, D),
                      Wfc, bfc.reshape(1, OUT))
    return out


# speedup vs baseline: 1.2446x; 1.0575x over previous
import functools

import jax
import jax.numpy as jnp
import numpy as np
from jax import lax
from jax.experimental import pallas as pl
from jax.experimental.pallas import tpu as pltpu
from jax.experimental.pallas import tpu_sc as plsc

N = 10000
E = 320000
D = 128
HD = 64
OUT = 64

NC = 2
NS = 16
NW = NC * NS

C = 128
KC = (E + NS * C - 1) // (NS * C)
KC = 160
E_PAD = NS * KC * C
N_ACC = 10112
ROWS_PER_TILE = N_ACC // NS

DEG_LEN = N + 240
DEG_PER_TILE = DEG_LEN // NS

_MESH = plsc.VectorSubcoreMesh(
    core_axis_name="c", subcore_axis_name="s", num_cores=NC, num_subcores=NS
)


@functools.partial(
    pl.kernel,
    out_type=jax.ShapeDtypeStruct((NC, DEG_LEN), jnp.float32),
    mesh=_MESH,
    scratch_types=[
        pltpu.VMEM((KC, C), jnp.int32),
        pltpu.VMEM((C,), jnp.float32),
        pltpu.VMEM((DEG_PER_TILE,), jnp.float32),
        pltpu.VMEM_SHARED((DEG_LEN,), jnp.float32),
    ],
    compiler_params=pltpu.CompilerParams(use_tc_tiling_on_sc=False),
)
def _deg_kernel(idx_hbm, out_hbm, idx_v, ones_v, stage_v, acc):
    c = lax.axis_index("c")
    s = lax.axis_index("s")

    one = jnp.ones((16,), jnp.float32)
    zero = jnp.zeros((16,), jnp.float32)
    for j in range(C // 16):
        ones_v[pl.ds(j * 16, 16)] = one

    @pl.loop(0, DEG_PER_TILE // 16)
    def _(r):
        stage_v[pl.ds(r * 16, 16)] = zero

    pltpu.sync_copy(stage_v, acc.at[pl.ds(s * DEG_PER_TILE, DEG_PER_TILE)])
    pltpu.sync_copy(idx_hbm.at[c, s], idx_v)
    plsc.subcore_barrier()

    @pl.loop(0, KC)
    def _(j):
        pltpu.sync_copy(ones_v, acc.at[idx_v.at[j]], add=True)

    plsc.subcore_barrier()
    pltpu.sync_copy(acc.at[pl.ds(s * DEG_PER_TILE, DEG_PER_TILE)], stage_v)
    pltpu.sync_copy(stage_v, out_hbm.at[c, pl.ds(s * DEG_PER_TILE, DEG_PER_TILE)])


NBUF = 5
GD = 4


@functools.partial(
    pl.kernel,
    out_type=jax.ShapeDtypeStruct((N_ACC, D), jnp.float32),
    mesh=_MESH,
    scratch_types=[
        pltpu.VMEM((KC, C), jnp.int32),
        pltpu.VMEM((KC, C), jnp.int32),
        [pltpu.VMEM((C, HD), jnp.float32)] * NBUF,
        [pltpu.SemaphoreType.DMA] * NBUF,
        pltpu.VMEM_SHARED((N_ACC, HD), jnp.float32),
    ],
    compiler_params=pltpu.CompilerParams(use_tc_tiling_on_sc=False),
)
def _prop_kernel(table_hbm, idx_hbm, out_hbm,
                 idx_s, idx_d, rows, sem_g, acc):
    c = lax.axis_index("c")
    s = lax.axis_index("s")

    zero = jnp.zeros((16,), jnp.float32)

    @pl.loop(0, C)
    def _(r):
        for j in range(HD // 16):
            rows[0][r, pl.ds(j * 16, 16)] = zero

    base = s * ROWS_PER_TILE
    n_full, rem = divmod(ROWS_PER_TILE, C)
    for i in range(n_full):
        pltpu.sync_copy(rows[0], acc.at[pl.ds(base + i * C, C)])
    if rem:
        pltpu.sync_copy(rows[0].at[pl.ds(0, rem)],
                        acc.at[pl.ds(base + n_full * C, rem)])
    pltpu.sync_copy(idx_hbm.at[0, s], idx_s)
    pltpu.sync_copy(idx_hbm.at[1, s], idx_d)

    @pl.loop(0, KC)
    def _(k):
        for jj in range(C // 16):
            v = idx_s[k, pl.ds(jj * 16, 16)]
            idx_s[k, pl.ds(jj * 16, 16)] = v + v + c

    plsc.subcore_barrier()

    def _gather(chunk, b):
        pltpu.async_copy(table_hbm.at[idx_s.at[chunk]], rows[b], sem_g[b])

    def _wait_gather(chunk, b):
        pltpu.make_async_copy(table_hbm.at[idx_s.at[chunk]], rows[b],
                              sem_g[b]).wait()

    for j in range(GD):
        _gather(j, j)

    @pl.loop(0, KC, step=NBUF)
    def _(g):
        for j in range(NBUF):
            _wait_gather(g + j, j)
            pltpu.sync_copy(rows[j], acc.at[idx_d.at[g + j]], add=True)

            @pl.when(g + j + GD < KC)
            def _():
                _gather(g + j + GD, (j + GD) % NBUF)

    plsc.subcore_barrier()

    col = c * HD
    for i in range(n_full):
        pltpu.sync_copy(acc.at[pl.ds(base + i * C, C)], rows[0])
        pltpu.sync_copy(rows[0],
                        out_hbm.at[pl.ds(base + i * C, C), pl.ds(col, HD)])
    if rem:
        pltpu.sync_copy(acc.at[pl.ds(base + n_full * C, rem)],
                        rows[0].at[pl.ds(0, rem)])
        pltpu.sync_copy(rows[0].at[pl.ds(0, rem)],
                        out_hbm.at[pl.ds(base + n_full * C, rem),
                                   pl.ds(col, HD)])


_RB = 2000
_GRID = N // _RB


def _norm_from(deg_ref):
    d = deg_ref[...]
    return jnp.where(d > 0.0, lax.rsqrt(d), 0.0)


def _scale_body(x_ref, dout_ref, o_ref):
    o_ref[...] = x_ref[...] * _norm_from(dout_ref)


def _layer_body(p_ref, din_ref, dout_ref, w_ref, b_ref, o_ref):
    agg = p_ref[...] * _norm_from(din_ref)
    h = jnp.dot(agg, w_ref[...], preferred_element_type=jnp.float32)
    h = jax.nn.sigmoid(h + b_ref[...])
    o_ref[...] = h * _norm_from(dout_ref)


def _final_body(q_ref, din_ref, w2_ref, b2_ref, wfc_ref, bfc_ref, o_ref):
    agg = q_ref[...] * _norm_from(din_ref)
    h = jnp.dot(agg, w2_ref[...], preferred_element_type=jnp.float32)
    h = jax.nn.sigmoid(h + b2_ref[...])
    o_ref[...] = jnp.dot(h, wfc_ref[...],
                         preferred_element_type=jnp.float32) + bfc_ref[...]


def _deg_spec():
    return pl.BlockSpec((_RB, 1), lambda i: (i, 0))


def _row_spec(w):
    return pl.BlockSpec((_RB, w), lambda i: (i, 0))


def _full_spec(shape):
    return pl.BlockSpec(shape, lambda i: tuple(0 for _ in shape))


_scale_call = pl.pallas_call(
    _scale_body,
    grid=(_GRID,),
    in_specs=[
        _row_spec(D),
        _deg_spec(),
    ],
    out_specs=_row_spec(D),
    out_shape=jax.ShapeDtypeStruct((N, D), jnp.float32),
)

_layer_call = pl.pallas_call(
    _layer_body,
    grid=(_GRID,),
    in_specs=[
        _row_spec(D),
        _deg_spec(),
        _deg_spec(),
        _full_spec((D, D)),
        _full_spec((1, D)),
    ],
    out_specs=_row_spec(D),
    out_shape=jax.ShapeDtypeStruct((N, D), jnp.float32),
)

_final_call = pl.pallas_call(
    _final_body,
    grid=(_GRID,),
    in_specs=[
        _row_spec(D),
        _deg_spec(),
        _full_spec((D, D)),
        _full_spec((1, D)),
        _full_spec((D, OUT)),
        _full_spec((1, OUT)),
    ],
    out_specs=pl.BlockSpec((_RB, OUT), lambda i: (i, 0)),
    out_shape=jax.ShapeDtypeStruct((N, OUT), jnp.float32),
)


_PAD = E_PAD - E
_PAD_SRC_HIST = np.bincount((np.arange(_PAD) * 131) % N,
                            minlength=N).astype(np.float32)


def kernel(x, edge_index, W1, b1, W2, b2, Wfc, bfc):
    ar = jnp.arange(_PAD, dtype=jnp.int32)
    pads = jnp.stack([(ar * 131) % N, N + (ar % NS)])
    idx = jnp.concatenate([edge_index.astype(jnp.int32), pads],
                          axis=1).reshape(2, NS, KC, C)

    degp = _deg_kernel(idx)
    dout = (degp[0, :N] - jnp.asarray(_PAD_SRC_HIST)).reshape(N, 1)
    din = degp[1, :N].reshape(N, 1)

    xs = _scale_call(x, dout)
    p = _prop_kernel(xs.reshape(2 * N, HD), idx)
    t2 = _layer_call(p, din, dout, W1, b1.reshape(1, D))
    q = _prop_kernel(t2.reshape(2 * N, HD), idx)
    out = _final_call(q, din, W2, b2.reshape(1, D),
                      Wfc, bfc.reshape(1, OUT))
    return out
